# baseline (speedup 1.0000x reference)
import functools

import jax
import jax.numpy as jnp
from jax import lax
from jax.experimental import pallas as pl
from jax.experimental.pallas import tpu as pltpu
from jax.experimental.pallas import tpu_sc as plsc

N = 10000
E = 160000
EE = E + N
H_IN = 4
HID = 256
OUT_CH = 64

BN = 400

NC = 2
NS = 16
NW = NC * NS
EB = 64
NB = (EE + NW * EB - 1) // (NW * EB)
NB0 = 112
NB1 = 2 * NB - NB0
NBMX = max(NB0, NB1)
TOTB = NS * (NB0 + NB1)
EP = TOTB * EB
NPAD = 10240
STRIPE = NPAD // NS


def _matmul_sliced(x, w_aug):
    n, K = x.shape
    M = w_aug.shape[1]

    def body(x_ref, w_ref, o_ref):
        o_ref[0] = jnp.dot(x_ref[...], w_ref[...],
                           preferred_element_type=jnp.float32)

    return pl.pallas_call(
        body,
        grid=(n // BN, M // 128),
        in_specs=[
            pl.BlockSpec((BN, K), lambda i, j: (i, 0)),
            pl.BlockSpec((K, 128), lambda i, j: (0, j)),
        ],
        out_specs=pl.BlockSpec((1, BN, 128), lambda i, j: (j, i, 0)),
        out_shape=jax.ShapeDtypeStruct((M // 128, n, 128), jnp.float32),
    )(x, w_aug)


def _edge_aggregate_sc(h_flat, srcp, dstp, n_sl):
    heads = (n_sl * 128) // 256 if n_sl > 1 else 1
    mesh = plsc.VectorSubcoreMesh(core_axis_name="c", subcore_axis_name="s")

    @functools.partial(
        pl.kernel,
        out_type=[
            jax.ShapeDtypeStruct((2 * (n_sl + 1) * NPAD, 128), jnp.float32),
            jax.ShapeDtypeStruct((TOTB * heads * EB,), jnp.float32),
        ],
        mesh=mesh,
        compiler_params=pltpu.CompilerParams(needs_layout_passes=False),
        scratch_types=[
            pltpu.VMEM((NBMX, EB), jnp.int32),
            pltpu.VMEM((NBMX, EB), jnp.int32),
            pltpu.VMEM((EB,), jnp.int32),
            pltpu.VMEM((EB,), jnp.int32),
            pltpu.VMEM((heads * EB,), jnp.float32),
            pltpu.VMEM((EB,), jnp.float32),
            pltpu.VMEM((EB,), jnp.float32),
            pltpu.VMEM((EB, 128), jnp.float32),
            pltpu.VMEM((EB, 128), jnp.float32),
            pltpu.VMEM_SHARED((NPAD, 128), jnp.float32),
            pltpu.SemaphoreType.DMA,
            pltpu.SemaphoreType.DMA,
            pltpu.SemaphoreType.DMA,
            pltpu.SemaphoreType.DMA,
            pltpu.SemaphoreType.DMA,
            pltpu.SemaphoreType.DMA,
        ],
    )
    def k(h_hbm, srcp_hbm, dstp_hbm, zeros_hbm, out_hbm, w_hbm,
          src_t, dst_t, adj0, adj1, wall_v, w0, w1, rows0, rows1,
          acc_sp, sg0, sg1, sw0, sw1, ss0, ss1):
        ci = lax.axis_index("c")
        si = lax.axis_index("s")
        nb = jnp.where(ci == 0, NB0, NB1)
        bstart = jnp.where(ci == 0, si * NB0, NS * NB0 + si * NB1)
        adj = (adj0, adj1)
        wv_ = (w0, w1)
        rows = (rows0, rows1)
        sg = (sg0, sg1)
        sw = (sw0, sw1)
        ss = (ss0, ss1)

        pltpu.sync_copy(srcp_hbm.at[pl.ds(bstart, NBMX)], src_t)
        pltpu.sync_copy(dstp_hbm.at[pl.ds(bstart, NBMX)], dst_t)

        lane = lax.broadcasted_iota(jnp.int32, (16,), 0)
        widx = jnp.minimum(lane, heads - 1) * EB
        wmask = lane < heads
        ebase = bstart * EB

        def zero_stripe():
            pltpu.sync_copy(zeros_hbm.at[pl.ds(si * STRIPE, STRIPE)],
                            acc_sp.at[pl.ds(si * STRIPE, STRIPE)])

        def copy_stripe_out(plane):
            off = (ci * (n_sl + 1) + plane) * NPAD + si * STRIPE
            pltpu.sync_copy(acc_sp.at[pl.ds(si * STRIPE, STRIPE)],
                            out_hbm.at[pl.ds(off, STRIPE)])

        zero_stripe()
        plsc.subcore_barrier()

        @pl.loop(0, nb)
        def _(b):
            for g in range(EB // 16):
                sv = src_t[b, pl.ds(g * 16, 16)]
                adj0[pl.ds(g * 16, 16)] = sv + n_sl * N
                dv = dst_t[b, pl.ds(g * 16, 16)]
                adj1[pl.ds(g * 16, 16)] = dv + (n_sl + 1) * N
            g1 = pltpu.async_copy(h_hbm.at[adj0], rows0, sg0)
            g2 = pltpu.async_copy(h_hbm.at[adj1], rows1, sg1)
            g1.wait()
            g2.wait()
            for i in range(EB):
                e = rows0[i, pl.ds(0, 16)] + rows1[i, pl.ds(0, 16)]
                e = jnp.maximum(e, 0.2 * e)
                w = jnp.exp(e)
                valid = (ebase + b * EB + i) < EE
                w = jnp.where(valid, w, jnp.zeros((16,), jnp.float32))
                rows1[i, pl.ds(0, 16)] = w
                plsc.store_scatter(wall_v, [widx + i], w, mask=wmask)
            pltpu.sync_copy(rows1, acc_sp.at[dst_t.at[b]], add=True)
            woff = (bstart + b) * heads * EB
            pltpu.sync_copy(wall_v, w_hbm.at[pl.ds(woff, heads * EB)])

        plsc.subcore_barrier()
        copy_stripe_out(0)

        def fire(b, u, srow_base, hd):
            for g in range(EB // 16):
                sv = src_t[b, pl.ds(g * 16, 16)]
                adj[u][pl.ds(g * 16, 16)] = sv + srow_base
            pltpu.async_copy(h_hbm.at[adj[u]], rows[u], sg[u])
            woff = ((bstart + b) * heads + hd) * EB
            pltpu.async_copy(w_hbm.at[pl.ds(woff, EB)], wv_[u], sw[u])

        def wait_gather(u):
            pltpu.make_async_copy(h_hbm.at[adj[u]], rows[u], sg[u]).wait()
            pltpu.make_async_copy(w_hbm.at[pl.ds(0, EB)], wv_[u], sw[u]).wait()

        def wait_scatter(u):
            pltpu.make_async_copy(rows[u], acc_sp.at[dst_t.at[0]],
                                  ss[u]).wait()

        def scale_and_scatter(b, u):
            for g in range(EB // 16):
                wv16 = wv_[u][pl.ds(g * 16, 16)]
                for l in range(16):
                    i = g * 16 + l
                    wvec = jnp.full((16,), wv16[l], jnp.float32)
                    for j in range(8):
                        rows[u][i, pl.ds(j * 16, 16)] = (
                            rows[u][i, pl.ds(j * 16, 16)] * wvec)
            pltpu.async_copy(rows[u], acc_sp.at[dst_t.at[b]], ss[u],
                             add=True)

        @pl.loop(0, n_sl)
        def _(s):
            hd = s // 2 if n_sl > 1 else s * 0
            zero_stripe()
            plsc.subcore_barrier()
            srow_base = s * N

            fire(0, 0, srow_base, hd)

            @pl.loop(0, nb, step=2)
            def _(b0):
                @pl.when(b0 > 0)
                def _():
                    wait_scatter(1)
                fire(b0 + 1, 1, srow_base, hd)
                wait_gather(0)
                scale_and_scatter(b0, 0)
                @pl.when(b0 + 2 < nb)
                def _():
                    wait_scatter(0)
                    fire(b0 + 2, 0, srow_base, hd)
                wait_gather(1)
                scale_and_scatter(b0 + 1, 1)

            wait_scatter(0)
            wait_scatter(1)
            plsc.subcore_barrier()
            copy_stripe_out(s + 1)

    zeros = jnp.zeros((NPAD, 128), jnp.float32)
    out, _ = k(h_flat, srcp, dstp, zeros)
    return out.reshape(2, n_sl + 1, NPAD, 128)


def _finalize_relu(numer, b, heads, out_ch):
    n_sl = numer.shape[1] - 1

    def body(num_ref, b_ref, o_ref):
        for s in range(n_sl):
            hd = s // 2
            val = num_ref[0, 1 + s, :, :] + num_ref[1, 1 + s, :, :]
            den = (num_ref[0, 0, :, hd:hd + 1]
                   + num_ref[1, 0, :, hd:hd + 1])
            sl = slice(s * 128, (s + 1) * 128)
            o_ref[:, sl] = jnp.maximum(val / den + b_ref[:, sl], 0.0)

    M = heads * out_ch
    return pl.pallas_call(
        body,
        grid=(N // BN,),
        in_specs=[
            pl.BlockSpec((2, n_sl + 1, BN, 128), lambda i: (0, 0, i, 0)),
            pl.BlockSpec((1, M), lambda i: (0, 0)),
        ],
        out_specs=pl.BlockSpec((BN, M), lambda i: (i, 0)),
        out_shape=jax.ShapeDtypeStruct((N, M), jnp.float32),
    )(numer, b.reshape(1, M))


def _finalize_logsoftmax(numer, b):
    def body(num_ref, b_ref, o_ref):
        den = num_ref[0, 0, :, 0:1] + num_ref[1, 0, :, 0:1]
        v = (num_ref[0, 1, :, :64] + num_ref[1, 1, :, :64]) / den + b_ref[...]
        z = v - jnp.max(v, axis=1, keepdims=True)
        o_ref[...] = z - jnp.log(jnp.sum(jnp.exp(z), axis=1, keepdims=True))

    return pl.pallas_call(
        body,
        grid=(N // BN,),
        in_specs=[
            pl.BlockSpec((2, 2, BN, 128), lambda i: (0, 0, i, 0)),
            pl.BlockSpec((1, OUT_CH), lambda i: (0, 0)),
        ],
        out_specs=pl.BlockSpec((BN, OUT_CH), lambda i: (i, 0)),
        out_shape=jax.ShapeDtypeStruct((N, OUT_CH), jnp.float32),
    )(numer, b.reshape(1, OUT_CH))


def _augment_w(W, a_s, a_d):
    K, M = W.shape
    heads, out_ch = a_s.shape
    n_sl = (M + 127) // 128
    As = jnp.zeros((M, heads), W.dtype)
    Ad = jnp.zeros((M, heads), W.dtype)
    for h in range(heads):
        sl = slice(h * out_ch, (h + 1) * out_ch)
        As = As.at[sl, h].set(a_s[h])
        Ad = Ad.at[sl, h].set(a_d[h])
    zc = jnp.zeros((K, 128 - heads), W.dtype)
    wp = jnp.zeros((K, n_sl * 128 - M), W.dtype)
    return jnp.concatenate([W, wp, W @ As, zc, W @ Ad, zc], axis=1)


def _gat_layer(x, srcp, dstp, W, a_s, a_d, heads, out_ch):
    M = heads * out_ch
    n_sl = (M + 127) // 128
    w_aug = _augment_w(W, a_s, a_d)
    out = _matmul_sliced(x, w_aug)
    h_flat = out.reshape((n_sl + 2) * N, 128)
    return _edge_aggregate_sc(h_flat, srcp, dstp, n_sl)


def kernel(x, edge_index, W1, a_s1, a_d1, b1, W2, a_s2, a_d2, b2,
           W3, a_s3, a_d3, b3):
    loop = jnp.arange(N, dtype=edge_index.dtype)
    pad = jnp.zeros((EP - EE,), edge_index.dtype)
    srcp = jnp.concatenate([edge_index[0], loop, pad]).reshape(TOTB, EB)
    dstp = jnp.concatenate([edge_index[1], loop, pad]).reshape(TOTB, EB)

    numer = _gat_layer(x, srcp, dstp, W1, a_s1, a_d1, H_IN, HID)
    h = _finalize_relu(numer, b1, H_IN, HID)
    numer = _gat_layer(h, srcp, dstp, W2, a_s2, a_d2, H_IN, HID)
    h = _finalize_relu(numer, b2, H_IN, HID)
    numer = _gat_layer(h, srcp, dstp, W3, a_s3, a_d3, 1, OUT_CH)
    return _finalize_logsoftmax(numer, b3)


# speedup vs baseline: 1.0169x; 1.0169x over previous
import functools

import jax
import jax.numpy as jnp
from jax import lax
from jax.experimental import pallas as pl
from jax.experimental.pallas import tpu as pltpu
from jax.experimental.pallas import tpu_sc as plsc

N = 10000
E = 160000
EE = E + N
H_IN = 4
HID = 256
OUT_CH = 64

BN = 400

NC = 2
NS = 16
NW = NC * NS
EB = 64
NB = (EE + NW * EB - 1) // (NW * EB)
NB0 = 112
NB1 = 2 * NB - NB0
NBMX = max(NB0, NB1)
TOTB = NS * (NB0 + NB1)
EP = TOTB * EB
NPAD = 10240
STRIPE = NPAD // NS
ZCH = 40
ZROWS = STRIPE // ZCH


def _matmul_sliced(x, w_aug):
    n, K = x.shape
    M = w_aug.shape[1]

    def body(x_ref, w_ref, o_ref):
        o_ref[0] = jnp.dot(x_ref[...], w_ref[...],
                           preferred_element_type=jnp.float32)

    return pl.pallas_call(
        body,
        grid=(n // BN, M // 128),
        in_specs=[
            pl.BlockSpec((BN, K), lambda i, j: (i, 0)),
            pl.BlockSpec((K, 128), lambda i, j: (0, j)),
        ],
        out_specs=pl.BlockSpec((1, BN, 128), lambda i, j: (j, i, 0)),
        out_shape=jax.ShapeDtypeStruct((M // 128, n, 128), jnp.float32),
    )(x, w_aug)


def _edge_aggregate_sc(h_flat, srcp, dstp, n_sl):
    heads = (n_sl * 128) // 256 if n_sl > 1 else 1
    mesh = plsc.VectorSubcoreMesh(core_axis_name="c", subcore_axis_name="s")

    @functools.partial(
        pl.kernel,
        out_type=[
            jax.ShapeDtypeStruct((2 * (n_sl + 1) * NPAD, 128), jnp.float32),
            jax.ShapeDtypeStruct((TOTB * heads * EB,), jnp.float32),
        ],
        mesh=mesh,
        compiler_params=pltpu.CompilerParams(needs_layout_passes=False),
        scratch_types=[
            pltpu.VMEM((NBMX, EB), jnp.int32),
            pltpu.VMEM((NBMX, EB), jnp.int32),
            pltpu.VMEM((EB,), jnp.int32),
            pltpu.VMEM((EB,), jnp.int32),
            pltpu.VMEM((heads * EB,), jnp.float32),
            pltpu.VMEM((EB,), jnp.float32),
            pltpu.VMEM((EB,), jnp.float32),
            pltpu.VMEM((EB, 128), jnp.float32),
            pltpu.VMEM((EB, 128), jnp.float32),
            pltpu.VMEM((ZROWS, 128), jnp.float32),
            pltpu.VMEM_SHARED((NPAD, 128), jnp.float32),
            pltpu.SemaphoreType.DMA,
            pltpu.SemaphoreType.DMA,
            pltpu.SemaphoreType.DMA,
            pltpu.SemaphoreType.DMA,
            pltpu.SemaphoreType.DMA,
            pltpu.SemaphoreType.DMA,
        ],
    )
    def k(h_hbm, srcp_hbm, dstp_hbm, out_hbm, w_hbm,
          src_t, dst_t, adj0, adj1, wall_v, w0, w1, rows0, rows1,
          zero_v, acc_sp, sg0, sg1, sw0, sw1, ss0, ss1):
        ci = lax.axis_index("c")
        si = lax.axis_index("s")
        nb = jnp.where(ci == 0, NB0, NB1)
        bstart = jnp.where(ci == 0, si * NB0, NS * NB0 + si * NB1)
        adj = (adj0, adj1)
        wv_ = (w0, w1)
        rows = (rows0, rows1)
        sg = (sg0, sg1)
        sw = (sw0, sw1)
        ss = (ss0, ss1)

        @pl.loop(0, ZROWS)
        def _(r):
            for j in range(8):
                zero_v[r, pl.ds(j * 16, 16)] = jnp.zeros((16,), jnp.float32)

        pltpu.sync_copy(srcp_hbm.at[pl.ds(bstart, NBMX)], src_t)
        pltpu.sync_copy(dstp_hbm.at[pl.ds(bstart, NBMX)], dst_t)

        lane = lax.broadcasted_iota(jnp.int32, (16,), 0)
        widx = jnp.minimum(lane, heads - 1) * EB
        wmask = lane < heads
        ebase = bstart * EB

        def zero_stripe():
            zs = [pltpu.async_copy(
                zero_v, acc_sp.at[pl.ds(si * STRIPE + z * ZROWS, ZROWS)],
                sg0) for z in range(ZCH)]
            for z in zs:
                z.wait()

        def copy_stripe_out(plane):
            off = (ci * (n_sl + 1) + plane) * NPAD + si * STRIPE
            pltpu.sync_copy(acc_sp.at[pl.ds(si * STRIPE, STRIPE)],
                            out_hbm.at[pl.ds(off, STRIPE)])

        zero_stripe()
        plsc.subcore_barrier()

        @pl.loop(0, nb)
        def _(b):
            for g in range(EB // 16):
                sv = src_t[b, pl.ds(g * 16, 16)]
                adj0[pl.ds(g * 16, 16)] = sv + n_sl * N
                dv = dst_t[b, pl.ds(g * 16, 16)]
                adj1[pl.ds(g * 16, 16)] = dv + (n_sl + 1) * N
            g1 = pltpu.async_copy(h_hbm.at[adj0], rows0, sg0)
            g2 = pltpu.async_copy(h_hbm.at[adj1], rows1, sg1)
            g1.wait()
            g2.wait()
            for i in range(EB):
                e = rows0[i, pl.ds(0, 16)] + rows1[i, pl.ds(0, 16)]
                e = jnp.maximum(e, 0.2 * e)
                w = jnp.exp(e)
                valid = (ebase + b * EB + i) < EE
                w = jnp.where(valid, w, jnp.zeros((16,), jnp.float32))
                rows1[i, pl.ds(0, 16)] = w
                plsc.store_scatter(wall_v, [widx + i], w, mask=wmask)
            pltpu.sync_copy(rows1, acc_sp.at[dst_t.at[b]], add=True)
            woff = (bstart + b) * heads * EB
            pltpu.sync_copy(wall_v, w_hbm.at[pl.ds(woff, heads * EB)])

        plsc.subcore_barrier()
        copy_stripe_out(0)

        def fire(b, u, srow_base, hd):
            for g in range(EB // 16):
                sv = src_t[b, pl.ds(g * 16, 16)]
                adj[u][pl.ds(g * 16, 16)] = sv + srow_base
            pltpu.async_copy(h_hbm.at[adj[u]], rows[u], sg[u])
            woff = ((bstart + b) * heads + hd) * EB
            pltpu.async_copy(w_hbm.at[pl.ds(woff, EB)], wv_[u], sw[u])

        def wait_gather(u):
            pltpu.make_async_copy(h_hbm.at[adj[u]], rows[u], sg[u]).wait()
            pltpu.make_async_copy(w_hbm.at[pl.ds(0, EB)], wv_[u], sw[u]).wait()

        def wait_scatter(u):
            pltpu.make_async_copy(rows[u], acc_sp.at[dst_t.at[0]],
                                  ss[u]).wait()

        def scale_and_scatter(b, u):
            for g in range(EB // 16):
                wv16 = wv_[u][pl.ds(g * 16, 16)]
                for l in range(16):
                    i = g * 16 + l
                    wvec = jnp.full((16,), wv16[l], jnp.float32)
                    for j in range(8):
                        rows[u][i, pl.ds(j * 16, 16)] = (
                            rows[u][i, pl.ds(j * 16, 16)] * wvec)
            pltpu.async_copy(rows[u], acc_sp.at[dst_t.at[b]], ss[u],
                             add=True)

        @pl.loop(0, n_sl)
        def _(s):
            hd = s // 2 if n_sl > 1 else s * 0
            zero_stripe()
            plsc.subcore_barrier()
            srow_base = s * N

            fire(0, 0, srow_base, hd)

            @pl.loop(0, nb, step=2)
            def _(b0):
                @pl.when(b0 > 0)
                def _():
                    wait_scatter(1)
                fire(b0 + 1, 1, srow_base, hd)
                wait_gather(0)
                scale_and_scatter(b0, 0)
                @pl.when(b0 + 2 < nb)
                def _():
                    wait_scatter(0)
                    fire(b0 + 2, 0, srow_base, hd)
                wait_gather(1)
                scale_and_scatter(b0 + 1, 1)

            wait_scatter(0)
            wait_scatter(1)
            plsc.subcore_barrier()
            copy_stripe_out(s + 1)

    out, _ = k(h_flat, srcp, dstp)
    return out.reshape(2, n_sl + 1, NPAD, 128)


def _finalize_relu(numer, b, heads, out_ch):
    n_sl = numer.shape[1] - 1

    def body(num_ref, b_ref, o_ref):
        for s in range(n_sl):
            hd = s // 2
            val = num_ref[0, 1 + s, :, :] + num_ref[1, 1 + s, :, :]
            den = (num_ref[0, 0, :, hd:hd + 1]
                   + num_ref[1, 0, :, hd:hd + 1])
            sl = slice(s * 128, (s + 1) * 128)
            o_ref[:, sl] = jnp.maximum(val / den + b_ref[:, sl], 0.0)

    M = heads * out_ch
    return pl.pallas_call(
        body,
        grid=(N // BN,),
        in_specs=[
            pl.BlockSpec((2, n_sl + 1, BN, 128), lambda i: (0, 0, i, 0)),
            pl.BlockSpec((1, M), lambda i: (0, 0)),
        ],
        out_specs=pl.BlockSpec((BN, M), lambda i: (i, 0)),
        out_shape=jax.ShapeDtypeStruct((N, M), jnp.float32),
    )(numer, b.reshape(1, M))


def _finalize_logsoftmax(numer, b):
    def body(num_ref, b_ref, o_ref):
        den = num_ref[0, 0, :, 0:1] + num_ref[1, 0, :, 0:1]
        v = (num_ref[0, 1, :, :64] + num_ref[1, 1, :, :64]) / den + b_ref[...]
        z = v - jnp.max(v, axis=1, keepdims=True)
        o_ref[...] = z - jnp.log(jnp.sum(jnp.exp(z), axis=1, keepdims=True))

    return pl.pallas_call(
        body,
        grid=(N // BN,),
        in_specs=[
            pl.BlockSpec((2, 2, BN, 128), lambda i: (0, 0, i, 0)),
            pl.BlockSpec((1, OUT_CH), lambda i: (0, 0)),
        ],
        out_specs=pl.BlockSpec((BN, OUT_CH), lambda i: (i, 0)),
        out_shape=jax.ShapeDtypeStruct((N, OUT_CH), jnp.float32),
    )(numer, b.reshape(1, OUT_CH))


def _augment_w(W, a_s, a_d):
    K, M = W.shape
    heads, out_ch = a_s.shape
    n_sl = (M + 127) // 128
    As = jnp.zeros((M, heads), W.dtype)
    Ad = jnp.zeros((M, heads), W.dtype)
    for h in range(heads):
        sl = slice(h * out_ch, (h + 1) * out_ch)
        As = As.at[sl, h].set(a_s[h])
        Ad = Ad.at[sl, h].set(a_d[h])
    zc = jnp.zeros((K, 128 - heads), W.dtype)
    wp = jnp.zeros((K, n_sl * 128 - M), W.dtype)
    return jnp.concatenate([W, wp, W @ As, zc, W @ Ad, zc], axis=1)


def _gat_layer(x, srcp, dstp, W, a_s, a_d, heads, out_ch):
    M = heads * out_ch
    n_sl = (M + 127) // 128
    w_aug = _augment_w(W, a_s, a_d)
    out = _matmul_sliced(x, w_aug)
    h_flat = out.reshape((n_sl + 2) * N, 128)
    return _edge_aggregate_sc(h_flat, srcp, dstp, n_sl)


def kernel(x, edge_index, W1, a_s1, a_d1, b1, W2, a_s2, a_d2, b2,
           W3, a_s3, a_d3, b3):
    loop = jnp.arange(N, dtype=edge_index.dtype)
    pad = jnp.zeros((EP - EE,), edge_index.dtype)
    srcp = jnp.concatenate([edge_index[0], loop, pad]).reshape(TOTB, EB)
    dstp = jnp.concatenate([edge_index[1], loop, pad]).reshape(TOTB, EB)

    numer = _gat_layer(x, srcp, dstp, W1, a_s1, a_d1, H_IN, HID)
    h = _finalize_relu(numer, b1, H_IN, HID)
    numer = _gat_layer(h, srcp, dstp, W2, a_s2, a_d2, H_IN, HID)
    h = _finalize_relu(numer, b2, H_IN, HID)
    numer = _gat_layer(h, srcp, dstp, W3, a_s3, a_d3, 1, OUT_CH)
    return _finalize_logsoftmax(numer, b3)


# speedup vs baseline: 1.0256x; 1.0086x over previous
import functools

import jax
import jax.numpy as jnp
from jax import lax
from jax.experimental import pallas as pl
from jax.experimental.pallas import tpu as pltpu
from jax.experimental.pallas import tpu_sc as plsc

N = 10000
E = 160000
EE = E + N
H_IN = 4
HID = 256
OUT_CH = 64

BN = 400

NC = 2
NS = 16
NW = NC * NS
EB = 64
NB = (EE + NW * EB - 1) // (NW * EB)
NB0 = 112
NB1 = 2 * NB - NB0
NBMX = max(NB0, NB1)
TOTB = NS * (NB0 + NB1)
TOTB_AL = TOTB + NBMX
EP = TOTB_AL * EB
NPAD = 10240
STRIPE = NPAD // NS
ZCH = 40
ZROWS = STRIPE // ZCH


def _matmul_sliced(x, w_aug):
    n, K = x.shape
    M = w_aug.shape[1]

    def body(x_ref, w_ref, o_ref):
        o_ref[0] = jnp.dot(x_ref[...], w_ref[...],
                           preferred_element_type=jnp.float32)

    return pl.pallas_call(
        body,
        grid=(n // BN, M // 128),
        in_specs=[
            pl.BlockSpec((BN, K), lambda i, j: (i, 0)),
            pl.BlockSpec((K, 128), lambda i, j: (0, j)),
        ],
        out_specs=pl.BlockSpec((1, BN, 128), lambda i, j: (j, i, 0)),
        out_shape=jax.ShapeDtypeStruct((M // 128, n, 128), jnp.float32),
    )(x, w_aug)


def _edge_aggregate_sc(h_flat, srcp, dstp, n_sl):
    heads = (n_sl * 128) // 256 if n_sl > 1 else 1
    mesh = plsc.VectorSubcoreMesh(core_axis_name="c", subcore_axis_name="s")

    @functools.partial(
        pl.kernel,
        out_type=[
            jax.ShapeDtypeStruct((2 * (n_sl + 1) * NPAD, 128), jnp.float32),
            jax.ShapeDtypeStruct((TOTB * heads * EB,), jnp.float32),
        ],
        mesh=mesh,
        compiler_params=pltpu.CompilerParams(needs_layout_passes=False),
        scratch_types=[
            pltpu.VMEM((NBMX, EB), jnp.int32),
            pltpu.VMEM((NBMX, EB), jnp.int32),
            pltpu.VMEM((EB,), jnp.int32),
            pltpu.VMEM((EB,), jnp.int32),
            pltpu.VMEM((heads * EB,), jnp.float32),
            pltpu.VMEM((EB,), jnp.float32),
            pltpu.VMEM((EB,), jnp.float32),
            pltpu.VMEM((EB, 128), jnp.float32),
            pltpu.VMEM((EB, 128), jnp.float32),
            pltpu.VMEM((ZROWS, 128), jnp.float32),
            pltpu.VMEM_SHARED((NPAD, 128), jnp.float32),
            pltpu.SemaphoreType.DMA,
            pltpu.SemaphoreType.DMA,
            pltpu.SemaphoreType.DMA,
            pltpu.SemaphoreType.DMA,
            pltpu.SemaphoreType.DMA,
            pltpu.SemaphoreType.DMA,
        ],
    )
    def k(h_hbm, srcp_hbm, dstp_hbm, out_hbm, w_hbm,
          src_t, dst_t, adj0, adj1, wall_v, w0, w1, rows0, rows1,
          zero_v, acc_sp, sg0, sg1, sw0, sw1, ss0, ss1):
        ci = lax.axis_index("c")
        si = lax.axis_index("s")
        nb = jnp.where(ci == 0, NB0, NB1)
        bstart = jnp.where(ci == 0, si * NB0, NS * NB0 + si * NB1)
        adj = (adj0, adj1)
        wv_ = (w0, w1)
        rows = (rows0, rows1)
        sg = (sg0, sg1)
        sw = (sw0, sw1)
        ss = (ss0, ss1)

        @pl.loop(0, ZROWS)
        def _(r):
            for j in range(8):
                zero_v[r, pl.ds(j * 16, 16)] = jnp.zeros((16,), jnp.float32)

        pltpu.sync_copy(srcp_hbm.at[pl.ds(bstart, NBMX)], src_t)
        pltpu.sync_copy(dstp_hbm.at[pl.ds(bstart, NBMX)], dst_t)

        lane = lax.broadcasted_iota(jnp.int32, (16,), 0)
        widx = jnp.minimum(lane, heads - 1) * EB
        wmask = lane < heads
        ebase = bstart * EB

        def zero_stripe():
            zs = [pltpu.async_copy(
                zero_v, acc_sp.at[pl.ds(si * STRIPE + z * ZROWS, ZROWS)],
                sg0) for z in range(ZCH)]
            for z in zs:
                z.wait()

        def copy_stripe_out(plane):
            off = (ci * (n_sl + 1) + plane) * NPAD + si * STRIPE
            pltpu.sync_copy(acc_sp.at[pl.ds(si * STRIPE, STRIPE)],
                            out_hbm.at[pl.ds(off, STRIPE)])

        zero_stripe()
        plsc.subcore_barrier()

        @pl.loop(0, nb)
        def _(b):
            for g in range(EB // 16):
                sv = src_t[b, pl.ds(g * 16, 16)]
                adj0[pl.ds(g * 16, 16)] = sv + n_sl * N
                dv = dst_t[b, pl.ds(g * 16, 16)]
                adj1[pl.ds(g * 16, 16)] = dv + (n_sl + 1) * N
            g1 = pltpu.async_copy(h_hbm.at[adj0], rows0, sg0)
            g2 = pltpu.async_copy(h_hbm.at[adj1], rows1, sg1)
            g1.wait()
            g2.wait()
            for i in range(EB):
                e = rows0[i, pl.ds(0, 16)] + rows1[i, pl.ds(0, 16)]
                e = jnp.maximum(e, 0.2 * e)
                w = jnp.exp(e)
                valid = (ebase + b * EB + i) < EE
                w = jnp.where(valid, w, jnp.zeros((16,), jnp.float32))
                rows1[i, pl.ds(0, 16)] = w
                plsc.store_scatter(wall_v, [widx + i], w, mask=wmask)
            pltpu.sync_copy(rows1, acc_sp.at[dst_t.at[b]], add=True)
            woff = (bstart + b) * heads * EB
            pltpu.sync_copy(wall_v, w_hbm.at[pl.ds(woff, heads * EB)])

        plsc.subcore_barrier()
        copy_stripe_out(0)

        def fire(b, u, srow_base, hd):
            for g in range(EB // 16):
                sv = src_t[b, pl.ds(g * 16, 16)]
                adj[u][pl.ds(g * 16, 16)] = sv + srow_base
            pltpu.async_copy(h_hbm.at[adj[u]], rows[u], sg[u])
            woff = ((bstart + b) * heads + hd) * EB
            pltpu.async_copy(w_hbm.at[pl.ds(woff, EB)], wv_[u], sw[u])

        def wait_gather(u):
            pltpu.make_async_copy(h_hbm.at[adj[u]], rows[u], sg[u]).wait()
            pltpu.make_async_copy(w_hbm.at[pl.ds(0, EB)], wv_[u], sw[u]).wait()

        def wait_scatter(u):
            pltpu.make_async_copy(rows[u], acc_sp.at[dst_t.at[0]],
                                  ss[u]).wait()

        def scale_and_scatter(b, u):
            for g in range(EB // 16):
                wv16 = wv_[u][pl.ds(g * 16, 16)]
                for l in range(16):
                    i = g * 16 + l
                    wvec = jnp.full((16,), wv16[l], jnp.float32)
                    for j in range(8):
                        rows[u][i, pl.ds(j * 16, 16)] = (
                            rows[u][i, pl.ds(j * 16, 16)] * wvec)
            pltpu.async_copy(rows[u], acc_sp.at[dst_t.at[b]], ss[u],
                             add=True)

        @pl.loop(0, n_sl)
        def _(s):
            hd = s // 2 if n_sl > 1 else s * 0
            zero_stripe()
            plsc.subcore_barrier()
            srow_base = s * N

            fire(0, 0, srow_base, hd)

            @pl.loop(0, nb, step=2)
            def _(b0):
                @pl.when(b0 > 0)
                def _():
                    wait_scatter(1)
                fire(b0 + 1, 1, srow_base, hd)
                wait_gather(0)
                scale_and_scatter(b0, 0)
                @pl.when(b0 + 2 < nb)
                def _():
                    wait_scatter(0)
                    fire(b0 + 2, 0, srow_base, hd)
                wait_gather(1)
                scale_and_scatter(b0 + 1, 1)

            wait_scatter(0)
            wait_scatter(1)
            plsc.subcore_barrier()
            copy_stripe_out(s + 1)

    out, _ = k(h_flat, srcp, dstp)
    return out.reshape(2, n_sl + 1, NPAD, 128)


def _finalize_relu(numer, b, heads, out_ch):
    n_sl = numer.shape[1] - 1

    def body(num_ref, b_ref, o_ref):
        for s in range(n_sl):
            hd = s // 2
            val = num_ref[0, 1 + s, :, :] + num_ref[1, 1 + s, :, :]
            den = (num_ref[0, 0, :, hd:hd + 1]
                   + num_ref[1, 0, :, hd:hd + 1])
            sl = slice(s * 128, (s + 1) * 128)
            o_ref[:, sl] = jnp.maximum(val / den + b_ref[:, sl], 0.0)

    M = heads * out_ch
    return pl.pallas_call(
        body,
        grid=(N // BN,),
        in_specs=[
            pl.BlockSpec((2, n_sl + 1, BN, 128), lambda i: (0, 0, i, 0)),
            pl.BlockSpec((1, M), lambda i: (0, 0)),
        ],
        out_specs=pl.BlockSpec((BN, M), lambda i: (i, 0)),
        out_shape=jax.ShapeDtypeStruct((N, M), jnp.float32),
    )(numer, b.reshape(1, M))


def _finalize_logsoftmax(numer, b):
    def body(num_ref, b_ref, o_ref):
        den = num_ref[0, 0, :, 0:1] + num_ref[1, 0, :, 0:1]
        v = (num_ref[0, 1, :, :64] + num_ref[1, 1, :, :64]) / den + b_ref[...]
        z = v - jnp.max(v, axis=1, keepdims=True)
        o_ref[...] = z - jnp.log(jnp.sum(jnp.exp(z), axis=1, keepdims=True))

    return pl.pallas_call(
        body,
        grid=(N // BN,),
        in_specs=[
            pl.BlockSpec((2, 2, BN, 128), lambda i: (0, 0, i, 0)),
            pl.BlockSpec((1, OUT_CH), lambda i: (0, 0)),
        ],
        out_specs=pl.BlockSpec((BN, OUT_CH), lambda i: (i, 0)),
        out_shape=jax.ShapeDtypeStruct((N, OUT_CH), jnp.float32),
    )(numer, b.reshape(1, OUT_CH))


def _augment_w(W, a_s, a_d):
    K, M = W.shape
    heads, out_ch = a_s.shape
    n_sl = (M + 127) // 128
    As = jnp.zeros((M, heads), W.dtype)
    Ad = jnp.zeros((M, heads), W.dtype)
    for h in range(heads):
        sl = slice(h * out_ch, (h + 1) * out_ch)
        As = As.at[sl, h].set(a_s[h])
        Ad = Ad.at[sl, h].set(a_d[h])
    zc = jnp.zeros((K, 128 - heads), W.dtype)
    wp = jnp.zeros((K, n_sl * 128 - M), W.dtype)
    return jnp.concatenate([W, wp, W @ As, zc, W @ Ad, zc], axis=1)


def _gat_layer(x, srcp, dstp, W, a_s, a_d, heads, out_ch):
    M = heads * out_ch
    n_sl = (M + 127) // 128
    w_aug = _augment_w(W, a_s, a_d)
    out = _matmul_sliced(x, w_aug)
    h_flat = out.reshape((n_sl + 2) * N, 128)
    return _edge_aggregate_sc(h_flat, srcp, dstp, n_sl)


def kernel(x, edge_index, W1, a_s1, a_d1, b1, W2, a_s2, a_d2, b2,
           W3, a_s3, a_d3, b3):
    loop = jnp.arange(N, dtype=edge_index.dtype)
    pad = jnp.zeros((EP - EE,), edge_index.dtype)
    srcp = jnp.concatenate([edge_index[0], loop, pad]).reshape(TOTB_AL, EB)
    dstp = jnp.concatenate([edge_index[1], loop, pad]).reshape(TOTB_AL, EB)

    numer = _gat_layer(x, srcp, dstp, W1, a_s1, a_d1, H_IN, HID)
    h = _finalize_relu(numer, b1, H_IN, HID)
    numer = _gat_layer(h, srcp, dstp, W2, a_s2, a_d2, H_IN, HID)
    h = _finalize_relu(numer, b2, H_IN, HID)
    numer = _gat_layer(h, srcp, dstp, W3, a_s3, a_d3, 1, OUT_CH)
    return _finalize_logsoftmax(numer, b3)
